# baseline (speedup 1.0000x reference)
import functools

import jax
import jax.numpy as jnp
from jax import lax
from jax.experimental import pallas as pl
from jax.experimental.pallas import tpu as pltpu
from jax.experimental.pallas import tpu_sc as plsc

N = 10000
F = 128
HID = 512
NC = 2
NS = 16
NW = NC * NS
CHUNK = 128
WB = 128
N_PAD = 10240
DPT = N_PAD // NS
RB = 1000


def _make_deg(e_pad):
    ept = e_pad // NW
    kch = ept // CHUNK
    mesh = plsc.VectorSubcoreMesh(core_axis_name="c", subcore_axis_name="s")

    @functools.partial(
        pl.kernel,
        out_type=jax.ShapeDtypeStruct((NC, N_PAD), jnp.float32),
        mesh=mesh,
        scratch_types=[
            pltpu.VMEM_SHARED((N_PAD,), jnp.float32),
            pltpu.VMEM((DPT,), jnp.float32),
            pltpu.VMEM((CHUNK,), jnp.int32),
            pltpu.VMEM((CHUNK,), jnp.float32),
        ],
    )
    def deg_kernel(col_hbm, ew_hbm, zeros_hbm, out_hbm, deg_sh, bounce,
                   colbuf, ewbuf):
        cid = lax.axis_index("c")
        sid = lax.axis_index("s")
        tid = cid * NS + sid
        pltpu.sync_copy(zeros_hbm, bounce)
        pltpu.sync_copy(bounce, deg_sh.at[pl.ds(sid * DPT, DPT)])
        plsc.subcore_barrier()

        def body(k, carry):
            base = tid * ept + k * CHUNK
            pltpu.sync_copy(col_hbm.at[pl.ds(base, CHUNK)], colbuf)
            pltpu.sync_copy(ew_hbm.at[pl.ds(base, CHUNK)], ewbuf)
            pltpu.sync_copy(ewbuf, deg_sh.at[colbuf], add=True)
            return carry

        lax.fori_loop(0, kch, body, 0)
        plsc.subcore_barrier()
        pltpu.sync_copy(deg_sh.at[pl.ds(sid * DPT, DPT)], bounce)
        pltpu.sync_copy(bounce, out_hbm.at[cid, pl.ds(sid * DPT, DPT)])

    return deg_kernel


def _make_spmm(e_pad):
    ept = e_pad // NW
    kch = ept // CHUNK
    mesh = plsc.VectorSubcoreMesh(core_axis_name="c", subcore_axis_name="s")

    @functools.partial(
        pl.kernel,
        out_type=jax.ShapeDtypeStruct((NC, N_PAD, F), jnp.float32),
        mesh=mesh,
        scratch_types=[
            pltpu.VMEM_SHARED((N_PAD, F), jnp.float32),
            pltpu.VMEM((WB, F), jnp.float32),
            pltpu.VMEM((CHUNK,), jnp.int32),
            pltpu.VMEM((CHUNK,), jnp.int32),
            pltpu.VMEM((CHUNK,), jnp.float32),
            pltpu.VMEM((CHUNK, F), jnp.float32),
            pltpu.SemaphoreType.DMA,
        ],
    )
    def spmm_kernel(y_hbm, row_hbm, col_hbm, ew_hbm, zeros_hbm, out_hbm,
                    acc_sh, bounce, rowbuf, colbuf, ewbuf, rbuf, sem):
        cid = lax.axis_index("c")
        sid = lax.axis_index("s")
        tid = cid * NS + sid
        r0 = sid * (N_PAD // NS)
        pltpu.sync_copy(zeros_hbm, bounce)
        for m in range(5):
            pltpu.sync_copy(bounce, acc_sh.at[pl.ds(r0 + m * WB, WB)])
        plsc.subcore_barrier()

        def chunk_body(k, carry):
            base = tid * ept + k * CHUNK
            pltpu.sync_copy(row_hbm.at[pl.ds(base, CHUNK)], rowbuf)
            pltpu.sync_copy(col_hbm.at[pl.ds(base, CHUNK)], colbuf)
            pltpu.sync_copy(ew_hbm.at[pl.ds(base, CHUNK)], ewbuf)
            pltpu.async_copy(y_hbm.at[rowbuf], rbuf, sem).wait()

            def scale_body(g, c2):
                ewv = ewbuf[pl.ds(g * 16, 16)]
                for l in range(16):
                    w = ewv[l]
                    e = g * 16 + l
                    for j in range(F // 16):
                        sl = pl.ds(j * 16, 16)
                        rbuf[e, sl] = rbuf[e, sl] * w
                return c2

            lax.fori_loop(0, CHUNK // 16, scale_body, 0)
            pltpu.sync_copy(rbuf, acc_sh.at[colbuf], add=True)
            return carry

        lax.fori_loop(0, kch, chunk_body, 0)
        plsc.subcore_barrier()
        for m in range(5):
            pltpu.sync_copy(acc_sh.at[pl.ds(r0 + m * WB, WB)], bounce)
            pltpu.sync_copy(bounce, out_hbm.at[cid, pl.ds(r0 + m * WB, WB)])

    return spmm_kernel


def _dinv_from(degp):
    deg = degp[:, 0] + degp[:, 1] + 1.0
    dinv = jnp.where(deg > 0, lax.rsqrt(jnp.maximum(deg, 1e-12)), 0.0)
    return deg, dinv


def _tc_pre_body(degs_ref, degd_ref, degc_ref, xr_ref, xa_ref,
                 ys_ref, yd_ref, yc_ref):
    _, dinv_s = _dinv_from(degs_ref[...])
    _, dinv_d = _dinv_from(degd_ref[...])
    _, dinv_c = _dinv_from(degc_ref[...])
    ys_ref[...] = xr_ref[...] * dinv_s[:, None]
    yd_ref[...] = xr_ref[...] * dinv_d[:, None]
    yc_ref[...] = xa_ref[...] * dinv_c[:, None]


def _tc_mid_body(p_ref, x_ref, degp_ref, w1_ref, b1_ref, w2_ref,
                 ys_ref, ysc_ref):
    deg, dinv = _dinv_from(degp_ref[...])
    p = p_ref[0] + p_ref[1]
    agg = p * dinv[:, None] + x_ref[...] / deg[:, None]
    xs = jnp.maximum(
        jnp.dot(agg, w1_ref[...], preferred_element_type=jnp.float32)
        + b1_ref[...], 0.0)
    ys = jnp.dot(xs, w2_ref[...], preferred_element_type=jnp.float32)
    ys_ref[...] = ys
    ysc_ref[...] = ys * dinv[:, None]


def _tc_final_body(qs_ref, qd_ref, pc_ref, ys_ref, yd_ref, xa_ref,
                   degs_ref, degd_ref, degc_ref,
                   wp3_ref, bp3_ref, bs_ref, bd_ref,
                   wf1a_ref, wf1b_ref, bf1_ref,
                   wf2a_ref, wf2b_ref, bf2_ref,
                   xsim_ref, xdist_ref, fused_ref, fusedpro_ref, pro_ref):
    deg_s, dinv_s = _dinv_from(degs_ref[...])
    deg_d, dinv_d = _dinv_from(degd_ref[...])
    deg_c, dinv_c = _dinv_from(degc_ref[...])
    x_sim = ((qs_ref[0] + qs_ref[1]) * dinv_s[:, None]
             + ys_ref[...] / deg_s[:, None] + bs_ref[...])
    x_dist = ((qd_ref[0] + qd_ref[1]) * dinv_d[:, None]
              + yd_ref[...] / deg_d[:, None] + bd_ref[...])
    aggc = ((pc_ref[0] + pc_ref[1]) * dinv_c[:, None]
            + xa_ref[...] / deg_c[:, None])
    pro = (jnp.dot(aggc, wp3_ref[...], preferred_element_type=jnp.float32)
           + bp3_ref[...])
    fused = (jnp.dot(x_sim, wf1a_ref[...], preferred_element_type=jnp.float32)
             + jnp.dot(x_dist, wf1b_ref[...],
                       preferred_element_type=jnp.float32)
             + bf1_ref[...])
    fused_pro = (jnp.dot(fused, wf2a_ref[...],
                         preferred_element_type=jnp.float32)
                 + jnp.dot(pro, wf2b_ref[...],
                           preferred_element_type=jnp.float32)
                 + bf2_ref[...])
    xsim_ref[...] = x_sim
    xdist_ref[...] = x_dist
    fused_ref[...] = fused
    fusedpro_ref[...] = fused_pro
    pro_ref[...] = pro


def _row_spec(width):
    return pl.BlockSpec((RB, width), lambda i: (i, 0))


def _part_spec():
    return pl.BlockSpec((2, RB, F), lambda i: (0, i, 0))


def _deg_spec():
    return pl.BlockSpec((RB, 2), lambda i: (i, 0))


def _full_spec(shape):
    nd = len(shape)
    return pl.BlockSpec(shape, lambda i, _n=nd: (0,) * _n)


def _pad_edges(edge_index, ew, e_pad):
    e = ew.shape[0]
    row = edge_index[0]
    col = edge_index[1]
    pad = e_pad - e
    if pad:
        zi = jnp.zeros((pad,), dtype=row.dtype)
        row = jnp.concatenate([row, zi])
        col = jnp.concatenate([col, zi])
        ew = jnp.concatenate([ew, jnp.zeros((pad,), dtype=ew.dtype)])
    return row, col, ew


def _round_up(e, m):
    return ((e + m - 1) // m) * m


def kernel(x_RNA, x_ADT, sim_edge_index, sim_edge_weight, dist_edge_index,
           dist_edge_weight, common_edge_index, common_edge_weight,
           W_r1, b_r1, W_r2, b_r2, W_p3, b_p3, W_sim, b_sim, W_dist, b_dist,
           W_f1, b_f1, W_f2, b_f2):
    f32 = jnp.float32
    es_pad = _round_up(sim_edge_weight.shape[0], NW * CHUNK)
    ed_pad = _round_up(dist_edge_weight.shape[0], NW * CHUNK)
    ec_pad = _round_up(common_edge_weight.shape[0], NW * CHUNK)
    row_s, col_s, ew_s = _pad_edges(sim_edge_index, sim_edge_weight, es_pad)
    row_d, col_d, ew_d = _pad_edges(dist_edge_index, dist_edge_weight, ed_pad)
    row_c, col_c, ew_c = _pad_edges(common_edge_index, common_edge_weight,
                                    ec_pad)

    zeros_deg = jnp.zeros((DPT,), f32)
    zeros_row = jnp.zeros((WB, F), f32)

    deg_s = _make_deg(es_pad)(col_s, ew_s, zeros_deg)
    deg_d = _make_deg(ed_pad)(col_d, ew_d, zeros_deg)
    deg_c = _make_deg(ec_pad)(col_c, ew_c, zeros_deg)
    degT_s = jnp.swapaxes(deg_s, 0, 1)[:N]
    degT_d = jnp.swapaxes(deg_d, 0, 1)[:N]
    degT_c = jnp.swapaxes(deg_c, 0, 1)[:N]

    grid = (N // RB,)
    y_s, y_d, y_c = pl.pallas_call(
        _tc_pre_body,
        grid=grid,
        in_specs=[_deg_spec(), _deg_spec(), _deg_spec(),
                  _row_spec(F), _row_spec(F)],
        out_specs=[_row_spec(F), _row_spec(F), _row_spec(F)],
        out_shape=[jax.ShapeDtypeStruct((N, F), f32)] * 3,
    )(degT_s, degT_d, degT_c, x_RNA, x_ADT)

    spmm_sim = _make_spmm(es_pad)
    spmm_dist = _make_spmm(ed_pad)
    spmm_com = _make_spmm(ec_pad)

    p_s = spmm_sim(y_s, row_s, col_s, ew_s, zeros_row)
    p_d = spmm_dist(y_d, row_d, col_d, ew_d, zeros_row)
    p_c = spmm_com(y_c, row_c, col_c, ew_c, zeros_row)

    def mid(p, degT, w1, b1, w2):
        return pl.pallas_call(
            _tc_mid_body,
            grid=grid,
            in_specs=[_part_spec(), _row_spec(F), _deg_spec(),
                      _full_spec((F, HID)), _full_spec((1, HID)),
                      _full_spec((HID, F))],
            out_specs=[_row_spec(F), _row_spec(F)],
            out_shape=[jax.ShapeDtypeStruct((N, F), f32)] * 2,
        )(p, x_RNA, degT, w1, b1.reshape(1, HID), w2)

    ys, ysc = mid(p_s, degT_s, W_r1, b_r1, W_sim)
    yd, ydc = mid(p_d, degT_d, W_r2, b_r2, W_dist)

    q_s = spmm_sim(ysc, row_s, col_s, ew_s, zeros_row)
    q_d = spmm_dist(ydc, row_d, col_d, ew_d, zeros_row)

    outs = pl.pallas_call(
        _tc_final_body,
        grid=grid,
        in_specs=[_part_spec(), _part_spec(), _part_spec(),
                  _row_spec(F), _row_spec(F), _row_spec(F),
                  _deg_spec(), _deg_spec(), _deg_spec(),
                  _full_spec((F, F)), _full_spec((1, F)),
                  _full_spec((1, F)), _full_spec((1, F)),
                  _full_spec((F, F)), _full_spec((F, F)), _full_spec((1, F)),
                  _full_spec((F, F)), _full_spec((F, F)), _full_spec((1, F))],
        out_specs=[_row_spec(F)] * 5,
        out_shape=[jax.ShapeDtypeStruct((N, F), f32)] * 5,
    )(q_s, q_d, p_c, ys, yd, x_ADT, degT_s, degT_d, degT_c,
      W_p3, b_p3.reshape(1, F), b_sim.reshape(1, F), b_dist.reshape(1, F),
      W_f1[:F], W_f1[F:], b_f1.reshape(1, F),
      W_f2[:F], W_f2[F:], b_f2.reshape(1, F))
    x_sim, x_dist, fused, fused_pro, pro = outs
    return (x_sim, x_dist, fused, fused_pro, pro)
